# baseline (speedup 1.0000x reference)
import functools

import jax
import jax.numpy as jnp
from jax import lax
from jax.experimental import pallas as pl
from jax.experimental.pallas import tpu as pltpu
from jax.experimental.pallas import tpu_sc as plsc

B = 4096
NS = 26
VOCAB = 100000
D = 16
L = 50
ND = 13

NC = 2
NSUB = 16
NW = NC * NSUB
BW = B // NW
SP_ROWS = BW * NS
SQ_ROWS = BW * L
CH = 128
SP_CHUNKS = SP_ROWS // CH
SQ_CHUNKS = SQ_ROWS // CH
LPAD = 64


def _body(sp_idx_hbm, sp_tab_hbm, sq_idx_hbm, sq_pad_hbm, sq_tab_hbm,
          out_sp_hbm, out_pool_hbm,
          idx_sp_v, idx_sq_v, idx_p_v, rows_v, pooled_v, t0_v, sem):
    c = lax.axis_index("c")
    s = lax.axis_index("s")
    wid = s * NC + c

    pltpu.sync_copy(sp_idx_hbm.at[wid], idx_sp_v)
    pltpu.sync_copy(sq_idx_hbm.at[wid], idx_sq_v)
    pltpu.sync_copy(sq_pad_hbm.at[wid], idx_p_v)
    pltpu.sync_copy(sq_tab_hbm.at[pl.ds(0, 1)], t0_v)

    descs = []
    for j in range(SP_CHUNKS):
        descs.append(pltpu.async_copy(
            sp_tab_hbm.at[idx_sp_v.at[j]],
            rows_v.at[pl.ds(j * CH, CH)], sem))
    for d_ in descs:
        d_.wait()
    pltpu.sync_copy(rows_v.at[pl.ds(0, SP_ROWS)],
                    out_sp_hbm.at[pl.ds(wid * SP_ROWS, SP_ROWS)])

    descs = []
    for j in range(SQ_CHUNKS):
        descs.append(pltpu.async_copy(
            sq_tab_hbm.at[idx_sq_v.at[j]],
            rows_v.at[pl.ds(j * CH, CH)], sem))
    for d_ in descs:
        d_.wait()

    t0 = t0_v[0, :]

    def pool_one(b, _):
        acc = rows_v[b * L, :]
        for l in range(1, L):
            acc = acc + rows_v[b * L + l, :]
        nvec = jnp.zeros((D,), jnp.int32)
        for ch in range(LPAD // D):
            ids = idx_p_v[b, pl.ds(ch * D, D)]
            nvec = nvec + jnp.where(ids > 0, 1, 0).astype(jnp.int32)
        n = nvec[0]
        for i in range(1, D):
            n = n + nvec[i]
        nb = lax.broadcast_in_dim(n.astype(jnp.float32), (D,), ())
        pooled = (acc - (50.0 - nb) * t0) / jnp.maximum(nb, 1.0)
        pooled_v[b, :] = pooled
        return 0

    lax.fori_loop(0, BW, pool_one, 0)
    pltpu.sync_copy(pooled_v, out_pool_hbm.at[pl.ds(wid * BW, BW)])


@functools.partial(jax.jit, static_argnames=())
def _run(sp_idx_r, sp_tab, seq_g, seq_p, seq_table):
    mesh = plsc.VectorSubcoreMesh(core_axis_name="c", subcore_axis_name="s")
    f = pl.kernel(
        _body,
        out_type=[
            jax.ShapeDtypeStruct((NW * SP_ROWS, D), jnp.float32),
            jax.ShapeDtypeStruct((B, D), jnp.float32),
        ],
        mesh=mesh,
        compiler_params=pltpu.CompilerParams(use_tc_tiling_on_sc=False),
        scratch_types=[
            pltpu.VMEM((SP_CHUNKS, CH), jnp.int32),
            pltpu.VMEM((SQ_CHUNKS, CH), jnp.int32),
            pltpu.VMEM((BW, LPAD), jnp.int32),
            pltpu.VMEM((SQ_ROWS, D), jnp.float32),
            pltpu.VMEM((BW, D), jnp.float32),
            pltpu.VMEM((1, D), jnp.float32),
            pltpu.SemaphoreType.DMA,
        ],
    )
    return f(sp_idx_r, sp_tab, seq_g, seq_p, seq_table)


def kernel(sparse_idx, seq_idx, dense_x, sparse_tables, seq_table):
    si = sparse_idx.astype(jnp.int32) + (
        jnp.arange(NS, dtype=jnp.int32) * VOCAB)[None, :]
    sp_idx_r = si.reshape(NW, SP_CHUNKS, CH)
    sp_tab = sparse_tables.reshape(NS * VOCAB, D)
    qi = seq_idx.astype(jnp.int32)
    seq_g = qi.reshape(NW, SQ_CHUNKS, CH)
    seq_p = jnp.pad(qi, ((0, 0), (0, LPAD - L))).reshape(NW, BW, LPAD)
    out_sp, out_pool = _run(sp_idx_r, sp_tab, seq_g, seq_p, seq_table)
    return jnp.concatenate(
        [out_sp.reshape(B, NS * D), out_pool, dense_x.astype(jnp.float32)],
        axis=1)
